# baseline (speedup 1.0000x reference)
import functools

import jax
import jax.numpy as jnp
from jax import lax
from jax.experimental import pallas as pl
from jax.experimental.pallas import tpu as pltpu
from jax.experimental.pallas import tpu_sc as plsc

F32 = jnp.float32
CUTOFF_R = 5.0


def _ssp(v):
    return jnp.log(0.5 * jnp.exp(v) + 0.5)


def _in2f_kernel(x_ref, w_ref, o_ref):
    o_ref[...] = jnp.dot(x_ref[...], w_ref[...], preferred_element_type=F32)


def _in2f(x2d, w_in):
    m, k = x2d.shape
    f = w_in.shape[1]
    g = 8
    return pl.pallas_call(
        _in2f_kernel,
        grid=(g,),
        in_specs=[
            pl.BlockSpec((m // g, k), lambda i: (i, 0)),
            pl.BlockSpec((k, f), lambda i: (0, 0)),
        ],
        out_specs=pl.BlockSpec((m // g, f), lambda i: (i, 0)),
        out_shape=jax.ShapeDtypeStruct((m, f), F32),
    )(x2d, w_in)


def _sc_gather(y2d, idx3):
    info = plsc.get_sparse_core_info()
    nc, ns = info.num_cores, info.num_subcores
    nw = nc * ns
    nch, ch = idx3.shape[1], idx3.shape[2]
    fdim = y2d.shape[1]
    assert idx3.shape[0] == nw
    nbuf = 4
    assert nch % nbuf == 0
    mesh = plsc.VectorSubcoreMesh(core_axis_name="c", subcore_axis_name="s")
    e = nw * nch * ch

    @functools.partial(
        pl.kernel,
        mesh=mesh,
        out_type=jax.ShapeDtypeStruct((e, fdim), F32),
        scratch_types=[pltpu.VMEM((nch, ch), jnp.int32)]
        + [pltpu.VMEM((ch, fdim), F32) for _ in range(nbuf)]
        + [pltpu.SemaphoreType.DMA for _ in range(2 * nbuf)],
    )
    def gk(y_hbm, idx_hbm, out_hbm, idx_v, *bufs_and_sems):
        rows = bufs_and_sems[:nbuf]
        gsem = bufs_and_sems[nbuf:2 * nbuf]
        wsem = bufs_and_sems[2 * nbuf:]
        wid = lax.axis_index("s") * nc + lax.axis_index("c")
        pltpu.sync_copy(idx_hbm.at[wid], idx_v)
        for b in range(nbuf):
            pltpu.async_copy(y_hbm.at[idx_v.at[b]], rows[b], gsem[b])

        def body(g, carry):
            for b in range(nbuf):
                j = g * nbuf + b
                dst = out_hbm.at[pl.ds((wid * nch + j) * ch, ch)]
                pltpu.make_async_copy(y_hbm.at[idx_v.at[j]], rows[b],
                                      gsem[b]).wait()
                pltpu.async_copy(rows[b], dst, wsem[b])
                nxt = j + nbuf

                @pl.when(nxt < nch)
                def _():
                    pltpu.make_async_copy(rows[b], dst, wsem[b]).wait()
                    pltpu.async_copy(y_hbm.at[idx_v.at[nxt]], rows[b],
                                     gsem[b])

                @pl.when(nxt >= nch)
                def _():
                    pltpu.make_async_copy(rows[b], dst, wsem[b]).wait()

            return carry

        lax.fori_loop(0, nch // nbuf, body, 0)

    return gk(y2d, idx3)


def _fused_kernel(ng, kpg, n, f_ref, yg_ref, w1_ref, b1_ref, w2_ref, b2_ref,
                  wo_ref, bo_ref, wd_ref, bd_ref, o_ref, acc_ref):
    kg = pl.program_id(1)
    total = None
    for j in range(kpg):
        f2 = f_ref[0, :, j, :]
        h = lax.dot_general(f2, w1_ref[...], (((0,), (0,)), ((), ())),
                            preferred_element_type=F32) + b1_ref[...]
        w = (jnp.dot(_ssp(h), w2_ref[...], preferred_element_type=F32)
             + b2_ref[...])
        contrib = w * yg_ref[pl.ds(j * n, n), :]
        total = contrib if total is None else total + contrib

    @pl.when(kg == 0)
    def _():
        acc_ref[...] = total

    @pl.when(kg > 0)
    def _():
        acc_ref[...] += total

    @pl.when(kg == ng - 1)
    def _():
        agg = acc_ref[...]
        v = _ssp(jnp.dot(agg, wo_ref[...], preferred_element_type=F32)
                 + bo_ref[...])
        o_ref[...] = (jnp.dot(v, wd_ref[...], preferred_element_type=F32)
                      + bd_ref[...])


def _fused(f4, y_g, w1, b1, w2, b2, w_out, b_out, w_d, b_d):
    b, s, nbh, n = f4.shape
    kpg = 8
    ng = nbh // kpg
    ff = w_d.shape[0]
    body = functools.partial(_fused_kernel, ng, kpg, n)
    return pl.pallas_call(
        body,
        grid=(b, ng),
        in_specs=[
            pl.BlockSpec((1, s, kpg, n), lambda bi, kg: (bi, 0, kg, 0)),
            pl.BlockSpec((kpg * n, ff), lambda bi, kg: (bi * ng + kg, 0)),
            pl.BlockSpec((s, ff), lambda bi, kg: (0, 0)),
            pl.BlockSpec((1, ff), lambda bi, kg: (0, 0)),
            pl.BlockSpec((ff, ff), lambda bi, kg: (0, 0)),
            pl.BlockSpec((1, ff), lambda bi, kg: (0, 0)),
            pl.BlockSpec((ff, ff), lambda bi, kg: (0, 0)),
            pl.BlockSpec((1, ff), lambda bi, kg: (0, 0)),
            pl.BlockSpec((ff, ff), lambda bi, kg: (0, 0)),
            pl.BlockSpec((1, ff), lambda bi, kg: (0, 0)),
        ],
        out_specs=pl.BlockSpec((n, ff), lambda bi, kg: (bi, 0)),
        out_shape=jax.ShapeDtypeStruct((b * n, ff), F32),
        scratch_shapes=[pltpu.VMEM((n, ff), F32)],
    )(f4, y_g, w1, b1, w2, b2, w_out, b_out, w_d, b_d)


def kernel(x, r_ij, neighbors, neighbor_mask, f_ij, W1, b1, W2, b2,
           W_in, W_out, b_out, W_d, b_d):
    b, n, f = x.shape
    nbh = neighbors.shape[2]
    s = f_ij.shape[3]
    e = b * n * nbh

    y2d = _in2f(x.reshape(b * n, f), W_in)
    zero_row = b * n
    table = jnp.concatenate(
        [y2d, jnp.zeros((8, f), F32)], axis=0)

    nbr_t = jnp.transpose(neighbors, (0, 2, 1)).astype(jnp.int32)
    keep = (jnp.transpose(r_ij, (0, 2, 1)) <= CUTOFF_R) & (
        jnp.transpose(neighbor_mask, (0, 2, 1)) > 0)
    base = (jnp.arange(b, dtype=jnp.int32) * n)[:, None, None]
    flat_idx = jnp.where(keep, nbr_t + base, zero_row).reshape(-1)

    info = plsc.get_sparse_core_info()
    nw = info.num_cores * info.num_subcores
    ch = 128
    idx3 = flat_idx.reshape(nw, e // (nw * ch), ch)
    y_g = _sc_gather(table, idx3)

    f4 = jnp.transpose(f_ij, (0, 3, 2, 1))

    out = _fused(
        f4, y_g,
        W1, b1.reshape(1, -1), W2, b2.reshape(1, -1),
        W_out, b_out.reshape(1, -1), W_d, b_d.reshape(1, -1),
    )
    return out.reshape(b, n, f)


# speedup vs baseline: 9.4888x; 9.4888x over previous
import functools

import jax
import jax.numpy as jnp
from jax import lax
from jax.experimental import pallas as pl
from jax.experimental.pallas import tpu as pltpu
from jax.experimental.pallas import tpu_sc as plsc

F32 = jnp.float32
CUTOFF_R = 5.0


def _ssp(v):
    return jnp.log(0.5 * jnp.exp(v) + 0.5)


def _in2f_kernel(x_ref, w_ref, o_ref):
    o_ref[...] = jnp.dot(x_ref[...], w_ref[...], preferred_element_type=F32)


def _in2f(x2d, w_in):
    m, k = x2d.shape
    f = w_in.shape[1]
    g = 8
    return pl.pallas_call(
        _in2f_kernel,
        grid=(g,),
        in_specs=[
            pl.BlockSpec((m // g, k), lambda i: (i, 0)),
            pl.BlockSpec((k, f), lambda i: (0, 0)),
        ],
        out_specs=pl.BlockSpec((m // g, f), lambda i: (i, 0)),
        out_shape=jax.ShapeDtypeStruct((m, f), F32),
    )(x2d, w_in)


def _sc_gather(y2d, idx3):
    info = plsc.get_sparse_core_info()
    nc, ns = info.num_cores, info.num_subcores
    nw = nc * ns
    nch, ch = idx3.shape[1], idx3.shape[2]
    fdim = y2d.shape[1]
    assert idx3.shape[0] == nw
    nbuf = 4
    assert nch % nbuf == 0
    mesh = plsc.VectorSubcoreMesh(core_axis_name="c", subcore_axis_name="s")
    e = nw * nch * ch

    @functools.partial(
        pl.kernel,
        mesh=mesh,
        out_type=jax.ShapeDtypeStruct((e, fdim), F32),
        scratch_types=[pltpu.VMEM((nch, ch), jnp.int32)]
        + [pltpu.VMEM((ch, fdim), F32) for _ in range(nbuf)]
        + [pltpu.SemaphoreType.DMA for _ in range(2 * nbuf)],
    )
    def gk(y_hbm, idx_hbm, out_hbm, idx_v, *bufs_and_sems):
        rows = bufs_and_sems[:nbuf]
        gsem = bufs_and_sems[nbuf:2 * nbuf]
        wsem = bufs_and_sems[2 * nbuf:]
        wid = lax.axis_index("s") * nc + lax.axis_index("c")
        pltpu.sync_copy(idx_hbm.at[wid], idx_v)
        for b in range(nbuf):
            pltpu.async_copy(y_hbm.at[idx_v.at[b]], rows[b], gsem[b])

        def body(g, carry):
            for b in range(nbuf):
                j = g * nbuf + b
                dst = out_hbm.at[pl.ds((wid * nch + j) * ch, ch)]
                pltpu.make_async_copy(y_hbm.at[idx_v.at[j]], rows[b],
                                      gsem[b]).wait()
                pltpu.async_copy(rows[b], dst, wsem[b])
                nxt = j + nbuf

                @pl.when(nxt < nch)
                def _():
                    pltpu.make_async_copy(rows[b], dst, wsem[b]).wait()
                    pltpu.async_copy(y_hbm.at[idx_v.at[nxt]], rows[b],
                                     gsem[b])

                @pl.when(nxt >= nch)
                def _():
                    pltpu.make_async_copy(rows[b], dst, wsem[b]).wait()

            return carry

        lax.fori_loop(0, nch // nbuf, body, 0)

    return gk(y2d, idx3)


def _fused_kernel(ng, kpg, n, f_ref, yg_ref, c_ref, w1_ref, b1_ref, w2_ref,
                  b2_ref, wo_ref, bo_ref, wd_ref, bd_ref, o_ref, acc_ref):
    kg = pl.program_id(1)
    ct = jnp.transpose(c_ref[0])
    total = None
    for j in range(kpg):
        f2 = f_ref[0, :, j, :]
        h = lax.dot_general(f2, w1_ref[...], (((0,), (0,)), ((), ())),
                            preferred_element_type=F32) + b1_ref[...]
        w = (jnp.dot(_ssp(h), w2_ref[...], preferred_element_type=F32)
             + b2_ref[...])
        contrib = w * ct[:, j:j + 1] * yg_ref[pl.ds(j * n, n), :]
        total = contrib if total is None else total + contrib

    @pl.when(kg == 0)
    def _():
        acc_ref[...] = total

    @pl.when(kg > 0)
    def _():
        acc_ref[...] += total

    @pl.when(kg == ng - 1)
    def _():
        agg = acc_ref[...]
        v = _ssp(jnp.dot(agg, wo_ref[...], preferred_element_type=F32)
                 + bo_ref[...])
        o_ref[...] = (jnp.dot(v, wd_ref[...], preferred_element_type=F32)
                      + bd_ref[...])


def _fused(f4, y_g, c_t, w1, b1, w2, b2, w_out, b_out, w_d, b_d):
    b, s, nbh, n = f4.shape
    kpg = 8
    ng = nbh // kpg
    ff = w_d.shape[0]
    body = functools.partial(_fused_kernel, ng, kpg, n)
    return pl.pallas_call(
        body,
        grid=(b, ng),
        in_specs=[
            pl.BlockSpec((1, s, kpg, n), lambda bi, kg: (bi, 0, kg, 0)),
            pl.BlockSpec((kpg * n, ff), lambda bi, kg: (bi * ng + kg, 0)),
            pl.BlockSpec((1, kpg, n), lambda bi, kg: (bi, kg, 0)),
            pl.BlockSpec((s, ff), lambda bi, kg: (0, 0)),
            pl.BlockSpec((1, ff), lambda bi, kg: (0, 0)),
            pl.BlockSpec((ff, ff), lambda bi, kg: (0, 0)),
            pl.BlockSpec((1, ff), lambda bi, kg: (0, 0)),
            pl.BlockSpec((ff, ff), lambda bi, kg: (0, 0)),
            pl.BlockSpec((1, ff), lambda bi, kg: (0, 0)),
            pl.BlockSpec((ff, ff), lambda bi, kg: (0, 0)),
            pl.BlockSpec((1, ff), lambda bi, kg: (0, 0)),
        ],
        out_specs=pl.BlockSpec((n, ff), lambda bi, kg: (bi, 0)),
        out_shape=jax.ShapeDtypeStruct((b * n, ff), F32),
        scratch_shapes=[pltpu.VMEM((n, ff), F32)],
    )(f4, y_g, c_t, w1, b1, w2, b2, w_out, b_out, w_d, b_d)


def kernel(x, r_ij, neighbors, neighbor_mask, f_ij, W1, b1, W2, b2,
           W_in, W_out, b_out, W_d, b_d):
    b, n, f = x.shape
    nbh = neighbors.shape[2]
    s = f_ij.shape[3]
    e = b * n * nbh

    y2d = _in2f(x.reshape(b * n, f), W_in)

    nbr_t = jnp.transpose(neighbors, (0, 2, 1)).astype(jnp.int32)
    c_t = (jnp.transpose(r_ij, (0, 2, 1)) <= CUTOFF_R).astype(F32) * (
        jnp.transpose(neighbor_mask, (0, 2, 1)))
    base = (jnp.arange(b, dtype=jnp.int32) * n)[:, None, None]
    flat_idx = (nbr_t + base).reshape(-1)

    info = plsc.get_sparse_core_info()
    nw = info.num_cores * info.num_subcores
    ch = 128
    idx3 = flat_idx.reshape(nw, e // (nw * ch), ch)
    y_g = _sc_gather(y2d, idx3)

    f4 = jnp.transpose(f_ij, (0, 3, 2, 1))

    out = _fused(
        f4, y_g, c_t,
        W1, b1.reshape(1, -1), W2, b2.reshape(1, -1),
        W_out, b_out.reshape(1, -1), W_d, b_d.reshape(1, -1),
    )
    return out.reshape(b, n, f)


# speedup vs baseline: 10.1338x; 1.0680x over previous
import functools

import jax
import jax.numpy as jnp
from jax import lax
from jax.experimental import pallas as pl
from jax.experimental.pallas import tpu as pltpu
from jax.experimental.pallas import tpu_sc as plsc

F32 = jnp.float32
CUTOFF_R = 5.0


def _ssp(v):
    return jnp.log(0.5 * jnp.exp(v) + 0.5)


def _in2f_kernel(x_ref, w_ref, o_ref):
    o_ref[...] = jnp.dot(x_ref[...], w_ref[...], preferred_element_type=F32)


def _in2f(x2d, w_in):
    m, k = x2d.shape
    f = w_in.shape[1]
    g = 8
    return pl.pallas_call(
        _in2f_kernel,
        grid=(g,),
        in_specs=[
            pl.BlockSpec((m // g, k), lambda i: (i, 0)),
            pl.BlockSpec((k, f), lambda i: (0, 0)),
        ],
        out_specs=pl.BlockSpec((m // g, f), lambda i: (i, 0)),
        out_shape=jax.ShapeDtypeStruct((m, f), F32),
    )(x2d, w_in)


def _sc_gather(y2d, idx3):
    info = plsc.get_sparse_core_info()
    nc, ns = info.num_cores, info.num_subcores
    nw = nc * ns
    nch, ch = idx3.shape[1], idx3.shape[2]
    fdim = y2d.shape[1]
    assert idx3.shape[0] == nw
    nbuf = 4
    assert nch % nbuf == 0
    mesh = plsc.VectorSubcoreMesh(core_axis_name="c", subcore_axis_name="s")
    e = nw * nch * ch

    @functools.partial(
        pl.kernel,
        mesh=mesh,
        out_type=jax.ShapeDtypeStruct((e, fdim), F32),
        scratch_types=[pltpu.VMEM((nch, ch), jnp.int32)]
        + [pltpu.VMEM((ch, fdim), F32) for _ in range(nbuf)]
        + [pltpu.SemaphoreType.DMA for _ in range(2 * nbuf)],
    )
    def gk(y_hbm, idx_hbm, out_hbm, idx_v, *bufs_and_sems):
        rows = bufs_and_sems[:nbuf]
        gsem = bufs_and_sems[nbuf:2 * nbuf]
        wsem = bufs_and_sems[2 * nbuf:]
        wid = lax.axis_index("s") * nc + lax.axis_index("c")
        pltpu.sync_copy(idx_hbm.at[wid], idx_v)
        for b in range(nbuf):
            pltpu.async_copy(y_hbm.at[idx_v.at[b]], rows[b], gsem[b])

        def body(g, carry):
            for b in range(nbuf):
                j = g * nbuf + b
                dst = out_hbm.at[pl.ds((wid * nch + j) * ch, ch)]
                pltpu.make_async_copy(y_hbm.at[idx_v.at[j]], rows[b],
                                      gsem[b]).wait()
                pltpu.async_copy(rows[b], dst, wsem[b])
                nxt = j + nbuf

                @pl.when(nxt < nch)
                def _():
                    pltpu.make_async_copy(rows[b], dst, wsem[b]).wait()
                    pltpu.async_copy(y_hbm.at[idx_v.at[nxt]], rows[b],
                                     gsem[b])

                @pl.when(nxt >= nch)
                def _():
                    pltpu.make_async_copy(rows[b], dst, wsem[b]).wait()

            return carry

        lax.fori_loop(0, nch // nbuf, body, 0)

    return gk(y2d, idx3)


def _fused_kernel(ng, kpg, n, kg0, f_ref, yg_ref, c_ref, w1_ref, b1_ref,
                  w2_ref, b2_ref, o_ref, acc_ref):
    kg = pl.program_id(1)
    ct = jnp.transpose(c_ref[0])
    total = None
    for j in range(kpg):
        f2 = f_ref[0, :, j, :]
        h = lax.dot_general(f2, w1_ref[...], (((0,), (0,)), ((), ())),
                            preferred_element_type=F32) + b1_ref[...]
        w = (jnp.dot(_ssp(h), w2_ref[...], preferred_element_type=F32)
             + b2_ref[...])
        contrib = w * ct[:, j:j + 1] * yg_ref[pl.ds(j * n, n), :]
        total = contrib if total is None else total + contrib

    @pl.when(kg == 0)
    def _():
        acc_ref[...] = total

    @pl.when(kg > 0)
    def _():
        acc_ref[...] += total

    @pl.when(kg == ng - 1)
    def _():
        o_ref[...] = acc_ref[...]


def _fused_partial(f4, y_g, c_t, w1, b1, w2, b2, kg0, ng):
    b, s, nbh, n = f4.shape
    kpg = 8
    ff = w2.shape[1]
    body = functools.partial(_fused_kernel, ng, kpg, n, kg0)
    return pl.pallas_call(
        body,
        grid=(b, ng),
        in_specs=[
            pl.BlockSpec((1, s, kpg, n), lambda bi, kg: (bi, 0, kg0 + kg, 0)),
            pl.BlockSpec((kpg * n, ff), lambda bi, kg: (bi * ng + kg, 0)),
            pl.BlockSpec((1, kpg, n), lambda bi, kg: (bi, kg0 + kg, 0)),
            pl.BlockSpec((s, ff), lambda bi, kg: (0, 0)),
            pl.BlockSpec((1, ff), lambda bi, kg: (0, 0)),
            pl.BlockSpec((ff, ff), lambda bi, kg: (0, 0)),
            pl.BlockSpec((1, ff), lambda bi, kg: (0, 0)),
        ],
        out_specs=pl.BlockSpec((n, ff), lambda bi, kg: (bi, 0)),
        out_shape=jax.ShapeDtypeStruct((b * n, ff), F32),
        scratch_shapes=[pltpu.VMEM((n, ff), F32)],
    )(f4, y_g, c_t, w1, b1, w2, b2)


def _tail_kernel(nagg, wo_ref, bo_ref, wd_ref, bd_ref, *refs):
    aggs, o_ref = refs[:nagg], refs[nagg]
    agg = aggs[0][...]
    for a in aggs[1:]:
        agg += a[...]
    v = _ssp(jnp.dot(agg, wo_ref[...], preferred_element_type=F32)
             + bo_ref[...])
    o_ref[...] = (jnp.dot(v, wd_ref[...], preferred_element_type=F32)
                  + bd_ref[...])


def _tail(aggs, w_out, b_out, w_d, b_d, n):
    bn, ff = aggs[0].shape
    g = bn // n
    body = functools.partial(_tail_kernel, len(aggs))
    return pl.pallas_call(
        body,
        grid=(g,),
        in_specs=[
            pl.BlockSpec((ff, ff), lambda i: (0, 0)),
            pl.BlockSpec((1, ff), lambda i: (0, 0)),
            pl.BlockSpec((ff, ff), lambda i: (0, 0)),
            pl.BlockSpec((1, ff), lambda i: (0, 0)),
        ] + [pl.BlockSpec((n, ff), lambda i: (i, 0)) for _ in aggs],
        out_specs=pl.BlockSpec((n, ff), lambda i: (i, 0)),
        out_shape=jax.ShapeDtypeStruct((bn, ff), F32),
    )(w_out, b_out, w_d, b_d, *aggs)


def kernel(x, r_ij, neighbors, neighbor_mask, f_ij, W1, b1, W2, b2,
           W_in, W_out, b_out, W_d, b_d):
    b, n, f = x.shape
    nbh = neighbors.shape[2]
    s = f_ij.shape[3]
    e = b * n * nbh

    y2d = _in2f(x.reshape(b * n, f), W_in)

    nbr_t = jnp.transpose(neighbors, (0, 2, 1)).astype(jnp.int32)
    c_t = (jnp.transpose(r_ij, (0, 2, 1)) <= CUTOFF_R).astype(F32) * (
        jnp.transpose(neighbor_mask, (0, 2, 1)))
    base = (jnp.arange(b, dtype=jnp.int32) * n)[:, None, None]
    flat4 = nbr_t + base

    info = plsc.get_sparse_core_info()
    nw = info.num_cores * info.num_subcores
    ch = 128

    f4 = jnp.transpose(f_ij, (0, 3, 2, 1))

    nsplit = 4
    kc = nbh // nsplit
    b1r, b2r = b1.reshape(1, -1), b2.reshape(1, -1)
    aggs = []
    for ci in range(nsplit):
        idx_c = flat4[:, ci * kc:(ci + 1) * kc, :].reshape(
            nw, (b * kc * n) // (nw * ch), ch)
        y_g_c = _sc_gather(y2d, idx_c)
        aggs.append(_fused_partial(
            f4, y_g_c, c_t, W1, b1r, W2, b2r,
            kg0=ci * (kc // 8), ng=kc // 8))

    out = _tail(aggs, W_out, b_out.reshape(1, -1), W_d,
                b_d.reshape(1, -1), n)
    return out.reshape(b, n, f)
